# baseline (speedup 1.0000x reference)
import functools
import math

import jax
import jax.numpy as jnp
from jax import lax
from jax.experimental import pallas as pl
from jax.experimental.pallas import tpu as pltpu
from jax.experimental.pallas import tpu_sc as plsc

RANK = 32
K1 = 32
B = 4096
NUM_TRAIN = 1000000
BATCH_SIZE = 128
SIGMA = 1.0
C0 = -0.5 * math.log(2.0 * math.pi * SIGMA ** 2)
NROWS = 100000

NW = 32
B_PER_W = B // NW


@functools.cache
def _make_sc_gather():
    @functools.partial(
        pl.kernel,
        mesh=plsc.VectorSubcoreMesh(core_axis_name="c", subcore_axis_name="s"),
        out_type=jax.ShapeDtypeStruct((NW, RANK * B_PER_W), jnp.float32),
        scratch_types=[
            pltpu.VMEM((B_PER_W,), jnp.int32),
            pltpu.VMEM((RANK * B_PER_W,), jnp.int32),
            pltpu.VMEM((RANK * B_PER_W,), jnp.float32),
            pltpu.SemaphoreType.DMA,
            pltpu.SemaphoreType.DMA,
        ],
        compiler_params=pltpu.CompilerParams(use_tc_tiling_on_sc=False),
    )
    def _sc_gather(idx_hbm, tab, out_hbm, idx_v, fidx_v, rows_v, gsem, osem):
        wid = lax.axis_index("s") * 2 + lax.axis_index("c")
        base = wid * B_PER_W
        pltpu.sync_copy(idx_hbm.at[pl.ds(base, B_PER_W)], idx_v)
        for r in range(RANK):
            for c in range(B_PER_W // 16):
                vec = idx_v[pl.ds(c * 16, 16)]
                fidx_v[pl.ds(r * B_PER_W + c * 16, 16)] = vec + r * NROWS
        pltpu.async_copy(tab.at[fidx_v], rows_v, gsem).wait()
        pltpu.async_copy(rows_v, out_hbm.at[wid], osem).wait()

    return _sc_gather


@functools.cache
def _make_sc_gather_rows():
    @functools.partial(
        pl.kernel,
        mesh=plsc.VectorSubcoreMesh(core_axis_name="c", subcore_axis_name="s"),
        out_type=jax.ShapeDtypeStruct((B, RANK), jnp.float32),
        scratch_types=[
            pltpu.VMEM((B_PER_W,), jnp.int32),
            pltpu.VMEM((B_PER_W, RANK), jnp.float32),
            pltpu.SemaphoreType.DMA,
            pltpu.SemaphoreType.DMA,
        ],
        compiler_params=pltpu.CompilerParams(use_tc_tiling_on_sc=False),
    )
    def _sc_gather_rows(idx_hbm, tab, out_hbm, idx_v, rows_v, gsem, osem):
        wid = lax.axis_index("s") * 2 + lax.axis_index("c")
        base = wid * B_PER_W
        pltpu.sync_copy(idx_hbm.at[pl.ds(base, B_PER_W)], idx_v)
        pltpu.async_copy(tab.at[idx_v], rows_v, gsem).wait()
        pltpu.async_copy(rows_v, out_hbm.at[pl.ds(base, B_PER_W)], osem).wait()

    return _sc_gather_rows


BBL = 512
NBLK = B // BBL


def _tc_body(eps_ref, rows_ref, ys_ref, out_ref):
    i = pl.program_id(0)

    @pl.when(i == 0)
    def _init():
        out_ref[...] = jnp.full((1, 1), -(NUM_TRAIN / BATCH_SIZE) * B * C0,
                                dtype=jnp.float32)

    P = None
    for d in range(3):
        m = rows_ref[d]
        L = rows_ref[3 + d]
        s = m[None, :, :] + L[None, :, :] * eps_ref[d]
        P = s if P is None else P * s

    fs = jnp.sum(P, axis=1)
    dl = ys_ref[...] - fs
    d2 = jnp.sum(dl * dl)

    kl = jnp.float32(0.0)
    for d in range(3):
        m = rows_ref[d]
        L = rows_ref[3 + d]
        l2 = L * L
        kl = kl + jnp.sum(l2 + m * m - 1.0 - jnp.log(l2 + 1e-8))

    upd = (NUM_TRAIN / BATCH_SIZE) * d2 / (2.0 * K1) \
        + (0.5 / BATCH_SIZE) * kl
    out_ref[...] += jnp.full((1, 1), 1.0, dtype=jnp.float32) * upd


_tc_call = pl.pallas_call(
    _tc_body,
    grid=(NBLK,),
    in_specs=[
        pl.BlockSpec((3, K1, RANK, BBL), lambda i: (0, 0, 0, i)),
        pl.BlockSpec((6, RANK, BBL), lambda i: (0, 0, i)),
        pl.BlockSpec((1, BBL), lambda i: (0, i)),
    ],
    out_specs=pl.BlockSpec((1, 1), lambda i: (0, 0)),
    out_shape=jax.ShapeDtypeStruct((1, 1), jnp.float32),
    compiler_params=pltpu.CompilerParams(
        dimension_semantics=("arbitrary",),
    ),
)


def kernel(idx, ys, eps, mean0, mean1, mean2, chol0, chol1, chol2):
    idx_t = idx.T
    gather_rm = _make_sc_gather()
    gather_bm = _make_sc_gather_rows()
    m_w = [gather_rm(idx_t[d], t.T.reshape(-1))
           for d, t in enumerate((mean0, mean1, mean2))]
    c_w = [gather_bm(idx_t[d], t)
           for d, t in enumerate((chol0, chol1, chol2))]
    m_t = jnp.transpose(jnp.stack(m_w).reshape(3, NW, RANK, B_PER_W),
                        (0, 2, 1, 3)).reshape(3, RANK, B)
    c_t = jnp.transpose(jnp.stack(c_w), (0, 2, 1))
    rows_t = jnp.concatenate([m_t, c_t], axis=0)
    eps_t = jnp.transpose(eps, (0, 2, 3, 1))
    out = _tc_call(eps_t, rows_t, ys.reshape(1, B))
    return out.reshape(())


# speedup vs baseline: 1.4140x; 1.4140x over previous
import functools
import math

import jax
import jax.numpy as jnp
from jax import lax
from jax.experimental import pallas as pl
from jax.experimental.pallas import tpu as pltpu
from jax.experimental.pallas import tpu_sc as plsc

RANK = 32
K1 = 32
B = 4096
NUM_TRAIN = 1000000
BATCH_SIZE = 128
SIGMA = 1.0
C0 = -0.5 * math.log(2.0 * math.pi * SIGMA ** 2)
NROWS = 100000

NW = 32
B_PER_W = B // NW


@functools.cache
def _make_sc_gather():
    @functools.partial(
        pl.kernel,
        mesh=plsc.VectorSubcoreMesh(core_axis_name="c", subcore_axis_name="s"),
        out_type=jax.ShapeDtypeStruct((NW, RANK * B_PER_W), jnp.float32),
        scratch_types=[
            pltpu.VMEM((B_PER_W,), jnp.int32),
            pltpu.VMEM((RANK * B_PER_W,), jnp.int32),
            pltpu.VMEM((RANK * B_PER_W,), jnp.float32),
            pltpu.SemaphoreType.DMA,
            pltpu.SemaphoreType.DMA,
        ],
        compiler_params=pltpu.CompilerParams(use_tc_tiling_on_sc=False),
    )
    def _sc_gather(idx_hbm, tab, out_hbm, idx_v, fidx_v, rows_v, gsem, osem):
        wid = lax.axis_index("s") * 2 + lax.axis_index("c")
        base = wid * B_PER_W
        pltpu.sync_copy(idx_hbm.at[pl.ds(base, B_PER_W)], idx_v)
        for r in range(RANK):
            for c in range(B_PER_W // 16):
                vec = idx_v[pl.ds(c * 16, 16)]
                fidx_v[pl.ds(r * B_PER_W + c * 16, 16)] = vec + r * NROWS
        pltpu.async_copy(tab.at[fidx_v], rows_v, gsem).wait()
        pltpu.async_copy(rows_v, out_hbm.at[wid], osem).wait()

    return _sc_gather


BBL = 1024
NBLK = B // BBL


def _tc_body(eps_ref, rows_ref, ys_ref, out_ref):
    i = pl.program_id(0)

    @pl.when(i == 0)
    def _init():
        out_ref[...] = jnp.full((1, 1), -(NUM_TRAIN / BATCH_SIZE) * B * C0,
                                dtype=jnp.float32)

    P = None
    for d in range(3):
        m = rows_ref[d]
        L = rows_ref[3 + d]
        s = m[None, :, :] + L[None, :, :] * eps_ref[d]
        P = s if P is None else P * s

    fs = jnp.sum(P, axis=1)
    dl = ys_ref[...] - fs
    d2 = jnp.sum(dl * dl)

    kl = jnp.float32(0.0)
    for d in range(3):
        m = rows_ref[d]
        L = rows_ref[3 + d]
        l2 = L * L
        kl = kl + jnp.sum(l2 + m * m - 1.0 - jnp.log(l2 + 1e-8))

    upd = (NUM_TRAIN / BATCH_SIZE) * d2 / (2.0 * K1) \
        + (0.5 / BATCH_SIZE) * kl
    out_ref[...] += jnp.full((1, 1), 1.0, dtype=jnp.float32) * upd


_tc_call = pl.pallas_call(
    _tc_body,
    grid=(NBLK,),
    in_specs=[
        pl.BlockSpec((3, K1, RANK, BBL), lambda i: (0, 0, 0, i)),
        pl.BlockSpec((6, RANK, BBL), lambda i: (0, 0, i)),
        pl.BlockSpec((1, BBL), lambda i: (0, i)),
    ],
    out_specs=pl.BlockSpec((1, 1), lambda i: (0, 0)),
    out_shape=jax.ShapeDtypeStruct((1, 1), jnp.float32),
    compiler_params=pltpu.CompilerParams(
        dimension_semantics=("arbitrary",),
    ),
)


def kernel(idx, ys, eps, mean0, mean1, mean2, chol0, chol1, chol2):
    idx_t = idx.T
    tabs = (mean0, mean1, mean2, chol0, chol1, chol2)
    gather = _make_sc_gather()
    rows_w = [gather(idx_t[t % 3], tabs[t].T.reshape(-1)) for t in range(6)]
    rw = jnp.stack(rows_w)
    rows_t = jnp.transpose(rw.reshape(6, NW, RANK, B_PER_W),
                           (0, 2, 1, 3)).reshape(6, RANK, B)
    eps_t = jnp.transpose(eps, (0, 2, 3, 1))
    out = _tc_call(eps_t, rows_t, ys.reshape(1, B))
    return out.reshape(())


# speedup vs baseline: 1.4179x; 1.0028x over previous
import functools
import math

import jax
import jax.numpy as jnp
from jax import lax
from jax.experimental import pallas as pl
from jax.experimental.pallas import tpu as pltpu
from jax.experimental.pallas import tpu_sc as plsc

RANK = 32
K1 = 32
B = 4096
NUM_TRAIN = 1000000
BATCH_SIZE = 128
SIGMA = 1.0
C0 = -0.5 * math.log(2.0 * math.pi * SIGMA ** 2)
NROWS = 100000

NW = 32
B_PER_W = B // NW


@functools.cache
def _make_sc_gather():
    @functools.partial(
        pl.kernel,
        mesh=plsc.VectorSubcoreMesh(core_axis_name="c", subcore_axis_name="s"),
        out_type=jax.ShapeDtypeStruct((NW, RANK * B_PER_W), jnp.float32),
        scratch_types=[
            pltpu.VMEM((B_PER_W,), jnp.int32),
            pltpu.VMEM((RANK * B_PER_W,), jnp.int32),
            pltpu.VMEM((RANK * B_PER_W,), jnp.float32),
            pltpu.SemaphoreType.DMA,
            pltpu.SemaphoreType.DMA,
        ],
        compiler_params=pltpu.CompilerParams(use_tc_tiling_on_sc=False),
    )
    def _sc_gather(idx_hbm, tab, out_hbm, idx_v, fidx_v, rows_v, gsem, osem):
        wid = lax.axis_index("s") * 2 + lax.axis_index("c")
        base = wid * B_PER_W
        pltpu.sync_copy(idx_hbm.at[pl.ds(base, B_PER_W)], idx_v)
        for r in range(RANK):
            for c in range(B_PER_W // 16):
                vec = idx_v[pl.ds(c * 16, 16)]
                fidx_v[pl.ds(r * B_PER_W + c * 16, 16)] = vec + r * NROWS
        pltpu.async_copy(tab.at[fidx_v], rows_v, gsem).wait()
        pltpu.async_copy(rows_v, out_hbm.at[wid], osem).wait()

    return _sc_gather


BBL = 512
NBLK = B // BBL


def _tc_body(eps_ref, rows_ref, ys_ref, out_ref):
    i = pl.program_id(0)

    @pl.when(i == 0)
    def _init():
        out_ref[...] = jnp.full((1, 1), -(NUM_TRAIN / BATCH_SIZE) * B * C0,
                                dtype=jnp.float32)

    P = None
    for d in range(3):
        m = rows_ref[d]
        L = rows_ref[3 + d]
        s = m[None, :, :] + L[None, :, :] * eps_ref[d]
        P = s if P is None else P * s

    fs = jnp.sum(P, axis=1)
    dl = ys_ref[...] - fs
    d2 = jnp.sum(dl * dl)

    kl = jnp.float32(0.0)
    for d in range(3):
        m = rows_ref[d]
        L = rows_ref[3 + d]
        l2 = L * L
        kl = kl + jnp.sum(l2 + m * m - 1.0 - jnp.log(l2 + 1e-8))

    upd = (NUM_TRAIN / BATCH_SIZE) * d2 / (2.0 * K1) \
        + (0.5 / BATCH_SIZE) * kl
    out_ref[...] += jnp.full((1, 1), 1.0, dtype=jnp.float32) * upd


_tc_call = pl.pallas_call(
    _tc_body,
    grid=(NBLK,),
    in_specs=[
        pl.BlockSpec((3, K1, RANK, BBL), lambda i: (0, 0, 0, i)),
        pl.BlockSpec((6, RANK, BBL), lambda i: (0, 0, i)),
        pl.BlockSpec((1, BBL), lambda i: (0, i)),
    ],
    out_specs=pl.BlockSpec((1, 1), lambda i: (0, 0)),
    out_shape=jax.ShapeDtypeStruct((1, 1), jnp.float32),
    compiler_params=pltpu.CompilerParams(
        dimension_semantics=("arbitrary",),
    ),
)


def kernel(idx, ys, eps, mean0, mean1, mean2, chol0, chol1, chol2):
    idx_t = idx.T
    tabs = (mean0, mean1, mean2, chol0, chol1, chol2)
    gather = _make_sc_gather()
    rows_w = [gather(idx_t[t % 3], tabs[t].T.reshape(-1)) for t in range(6)]
    rw = jnp.stack(rows_w)
    rows_t = jnp.transpose(rw.reshape(6, NW, RANK, B_PER_W),
                           (0, 2, 1, 3)).reshape(6, RANK, B)
    eps_t = jnp.transpose(eps, (0, 2, 3, 1))
    out = _tc_call(eps_t, rows_t, ys.reshape(1, B))
    return out.reshape(())
